# baseline (speedup 1.0000x reference)
import functools

import jax
import jax.numpy as jnp
from jax.experimental import pallas as pl
from jax.experimental.pallas import tpu as pltpu

BM = 2048


def _mlp_body(cb_ref, ea_ref, ec_ref, wa_ref, w1_ref, w2_ref, w3_ref,
              wf_ref, bfc_ref, wr_ref, br_ref, out_ref,
              ta_ref, t1_ref, t2_ref, t3_ref):
    @pl.when(pl.program_id(0) == 0)
    def _precompute_tables():
        ta_ref[:] = jnp.dot(ea_ref[:], wa_ref[:],
                            preferred_element_type=jnp.float32).astype(jnp.bfloat16)
        t1_ref[:] = jnp.dot(ec_ref[:], w1_ref[:],
                            preferred_element_type=jnp.float32).astype(jnp.bfloat16)
        t2_ref[:] = jnp.dot(ec_ref[:], w2_ref[:],
                            preferred_element_type=jnp.float32).astype(jnp.bfloat16)
        t3_ref[:] = jnp.dot(ec_ref[:], w3_ref[:],
                            preferred_element_type=jnp.float32).astype(jnp.bfloat16)

    cb = cb_ref[:]
    cb16 = cb.astype(jnp.bfloat16)
    iot = jax.lax.broadcasted_iota(jnp.int32, (BM, 128), 1).astype(jnp.bfloat16)
    one = jnp.bfloat16(1.0)
    zero = jnp.bfloat16(0.0)
    oh_a = jnp.where(iot == cb16[:, 0:1], one, zero)
    oh_f = jnp.where(iot == cb16[:, 7:8], one, zero)
    oh_s = jnp.where(iot == cb16[:, 8:9], one, zero)
    oh_t = jnp.where(iot == cb16[:, 9:10], one, zero)

    acc = jnp.dot(oh_a, ta_ref[:], preferred_element_type=jnp.float32)
    acc += jnp.dot(oh_f, t1_ref[:], preferred_element_type=jnp.float32)
    acc += jnp.dot(oh_s, t2_ref[:], preferred_element_type=jnp.float32)
    acc += jnp.dot(oh_t, t3_ref[:], preferred_element_type=jnp.float32)
    acc += jnp.dot(cb16, wf_ref[:], preferred_element_type=jnp.float32)
    h = jnp.maximum(acc + bfc_ref[:], jnp.float32(0.0))
    out_ref[:] = jnp.dot(h.astype(jnp.bfloat16), wr_ref[:],
                         preferred_element_type=jnp.float32) + br_ref[:]


def kernel(content, emb_avg, emb_cate, W_fc, b_fc, W_res, b_res):
    B = content.shape[0]
    WfcT = W_fc.T
    wa = WfcT[0:16]
    w1 = WfcT[22:38]
    w2 = WfcT[38:54]
    w3 = WfcT[54:70]
    wf = jnp.zeros((29, 128), jnp.float32)
    wf = wf.at[1:7].set(WfcT[16:22])
    wf = wf.at[10:29].set(WfcT[70:89])
    ea_pad = jnp.zeros((128, 16), jnp.float32).at[:91].set(emb_avg)
    ec_pad = jnp.zeros((128, 16), jnp.float32).at[:90].set(emb_cate)

    grid = (B // BM,)
    full = lambda *s: pl.BlockSpec(s, lambda i: (0,) * len(s))
    out = pl.pallas_call(
        _mlp_body,
        grid=grid,
        in_specs=[
            pl.BlockSpec((BM, 29), lambda i: (i, 0)),
            full(128, 16), full(128, 16),
            full(16, 128), full(16, 128), full(16, 128), full(16, 128),
            full(29, 128), full(1, 128), full(128, 128), full(1, 128),
        ],
        out_specs=pl.BlockSpec((BM, 128), lambda i: (i, 0)),
        out_shape=jax.ShapeDtypeStruct((B, 128), jnp.float32),
        scratch_shapes=[pltpu.VMEM((128, 128), jnp.bfloat16)] * 4,
    )(content, ea_pad, ec_pad, wa, w1, w2, w3, wf.astype(jnp.bfloat16),
      b_fc.reshape(1, 128), W_res.T.astype(jnp.bfloat16), b_res.reshape(1, 128))
    return out


# speedup vs baseline: 1.0464x; 1.0464x over previous
import functools

import jax
import jax.numpy as jnp
from jax.experimental import pallas as pl
from jax.experimental.pallas import tpu as pltpu

BM = 4096


def _mlp_body(cb_ref, ea_ref, ec_ref, wa_ref, w1_ref, w2_ref, w3_ref,
              wf_ref, bfc_ref, wr_ref, br_ref, out_ref,
              ta_ref, t1_ref, t2_ref, t3_ref):
    @pl.when(pl.program_id(0) == 0)
    def _precompute_tables():
        ta_ref[:] = (jnp.dot(ea_ref[:], wa_ref[:],
                             preferred_element_type=jnp.float32)
                     + bfc_ref[:]).astype(jnp.bfloat16)
        t1_ref[:] = jnp.dot(ec_ref[:], w1_ref[:],
                            preferred_element_type=jnp.float32).astype(jnp.bfloat16)
        t2_ref[:] = jnp.dot(ec_ref[:], w2_ref[:],
                            preferred_element_type=jnp.float32).astype(jnp.bfloat16)
        t3_ref[:] = jnp.dot(ec_ref[:], w3_ref[:],
                            preferred_element_type=jnp.float32).astype(jnp.bfloat16)

    cb = cb_ref[:]
    cb16 = cb.astype(jnp.bfloat16)
    iot = jax.lax.broadcasted_iota(jnp.int32, (1, 128), 1).astype(jnp.bfloat16)
    one = jnp.bfloat16(1.0)
    zero = jnp.bfloat16(0.0)
    oh_a = jnp.where(iot == cb16[:, 0:1], one, zero)
    oh_f = jnp.where(iot == cb16[:, 7:8], one, zero)
    oh_s = jnp.where(iot == cb16[:, 8:9], one, zero)
    oh_t = jnp.where(iot == cb16[:, 9:10], one, zero)

    acc = jnp.dot(oh_a, ta_ref[:], preferred_element_type=jnp.float32)
    acc += jnp.dot(oh_f, t1_ref[:], preferred_element_type=jnp.float32)
    acc += jnp.dot(oh_s, t2_ref[:], preferred_element_type=jnp.float32)
    acc += jnp.dot(oh_t, t3_ref[:], preferred_element_type=jnp.float32)
    acc += jnp.dot(cb16, wf_ref[:], preferred_element_type=jnp.float32)
    h = jnp.maximum(acc, jnp.float32(0.0))
    out_ref[:] = jnp.dot(h.astype(jnp.bfloat16), wr_ref[:],
                         preferred_element_type=jnp.float32) + br_ref[:]


def kernel(content, emb_avg, emb_cate, W_fc, b_fc, W_res, b_res):
    B = content.shape[0]
    WfcT = W_fc.T
    wa = WfcT[0:16]
    w1 = WfcT[22:38]
    w2 = WfcT[38:54]
    w3 = WfcT[54:70]
    wf = jnp.zeros((29, 128), jnp.float32)
    wf = wf.at[1:7].set(WfcT[16:22])
    wf = wf.at[10:29].set(WfcT[70:89])
    ea_pad = jnp.zeros((128, 16), jnp.float32).at[:91].set(emb_avg)
    ec_pad = jnp.zeros((128, 16), jnp.float32).at[:90].set(emb_cate)

    grid = (B // BM,)
    full = lambda *s: pl.BlockSpec(s, lambda i: (0,) * len(s))
    out = pl.pallas_call(
        _mlp_body,
        grid=grid,
        in_specs=[
            pl.BlockSpec((BM, 29), lambda i: (i, 0)),
            full(128, 16), full(128, 16),
            full(16, 128), full(16, 128), full(16, 128), full(16, 128),
            full(29, 128), full(1, 128), full(128, 128), full(1, 128),
        ],
        out_specs=pl.BlockSpec((BM, 128), lambda i: (i, 0)),
        out_shape=jax.ShapeDtypeStruct((B, 128), jnp.float32),
        scratch_shapes=[pltpu.VMEM((128, 128), jnp.bfloat16)] * 4,
    )(content, ea_pad, ec_pad, wa, w1, w2, w3, wf.astype(jnp.bfloat16),
      b_fc.reshape(1, 128), W_res.T.astype(jnp.bfloat16), b_res.reshape(1, 128))
    return out
